# baseline (speedup 1.0000x reference)
import functools

import jax
import jax.numpy as jnp
import numpy as np
from jax import lax
from jax.experimental import pallas as pl
from jax.experimental.pallas import tpu as pltpu
from jax.experimental.pallas import tpu_sc as plsc

_P = 0.7
_MAX_DROPOUT_RATIO = 0.6
_B, _N = 32, 65536

_key = jax.random.key(42)
_key, _kgate = jax.random.split(_key)
_GATE_V = float(jax.random.uniform(_kgate, (1,))[0])
_DROPS = []
for _b_i in range(_B):
    _key, _k1, _k2 = jax.random.split(_key, 3)
    _ratio = float(jax.random.uniform(_k1, (1,))[0]) * _MAX_DROPOUT_RATIO
    _num = int(_ratio * _N)
    if _GATE_V < _P and _num > 0:
        _perm = jax.random.permutation(_k2, _N)[:_num]
        _DROPS.append(np.sort(np.asarray(_perm, dtype=np.int32)))
    else:
        _DROPS.append(np.zeros((0,), dtype=np.int32))

_C = 8192
_NCH = _N // _C
_PAD_ROW = _C

_cnt = np.zeros((_B, _NCH), dtype=np.int32)
_off = np.zeros((_B, _NCH), dtype=np.int32)
_seqs = []
for _b_i in range(_B):
    parts = []
    pos = 0
    d = _DROPS[_b_i]
    for _k_i in range(_NCH):
        loc = d[(d >= _k_i * _C) & (d < (_k_i + 1) * _C)] - _k_i * _C
        n = loc.shape[0]
        npad = (-n) % 16
        if npad:
            loc = np.concatenate([loc, np.full((npad,), _PAD_ROW, np.int32)])
        _cnt[_b_i, _k_i] = loc.shape[0]
        _off[_b_i, _k_i] = pos
        pos += loc.shape[0]
        parts.append(loc)
    _seqs.append(np.concatenate(parts) if parts else np.zeros((0,), np.int32))

_LMAX = max(16, max(s.shape[0] for s in _seqs))
_LMAX += (-_LMAX) % 8
_IDX_NP = np.full((_B, _LMAX), _PAD_ROW, dtype=np.int32)
for _b_i in range(_B):
    _IDX_NP[_b_i, : _seqs[_b_i].shape[0]] = _seqs[_b_i]
_TAB_NP = np.zeros((_B, 16), dtype=np.int32)
_TAB_NP[:, :_NCH] = _cnt
_TAB_NP[:, 8 : 8 + _NCH] = _off

_NC, _NS = 2, 16

_HAS_DROPS = any(s.shape[0] for s in _seqs)

_C3 = _C * 3


@functools.cache
def _build_sc_dropout():
    mesh = plsc.VectorSubcoreMesh(
        core_axis_name="c", subcore_axis_name="s", num_cores=_NC, num_subcores=_NS
    )

    @functools.partial(
        pl.kernel,
        out_type=jax.ShapeDtypeStruct((_B, _N * 3), jnp.float32),
        mesh=mesh,
        compiler_params=pltpu.CompilerParams(
            use_tc_tiling_on_sc=False, needs_layout_passes=False
        ),
        scratch_types=[
            pltpu.VMEM((_C3 + 8,), jnp.float32),
            pltpu.VMEM((_LMAX,), jnp.int32),
            pltpu.VMEM((16,), jnp.int32),
        ],
    )
    def sc_dropout(pc_hbm, idx_hbm, tab_hbm, out_hbm, buf, idxv, tabv):
        b = lax.axis_index("s") * _NC + lax.axis_index("c")
        pltpu.sync_copy(idx_hbm.at[b], idxv)
        pltpu.sync_copy(tab_hbm.at[b], tabv)
        tv = tabv[pl.ds(0, 16)]
        vx = vy = vz = None
        for k in range(_NCH):
            pltpu.sync_copy(pc_hbm.at[b, pl.ds(k * _C3, _C3)], buf.at[pl.ds(0, _C3)])
            if k == 0:
                r0 = buf[pl.ds(0, 16)]
                vx = jnp.full((16,), r0[0], jnp.float32)
                vy = jnp.full((16,), r0[1], jnp.float32)
                vz = jnp.full((16,), r0[2], jnp.float32)
            cnt = tv[k]
            off = tv[8 + k]

            def sbody(i, carry, off=off, vx=vx, vy=vy, vz=vz):
                rows = idxv[pl.ds(off + i * 16, 16)]
                e = rows * 3
                plsc.store_scatter(buf, [e], vx)
                plsc.store_scatter(buf, [e + 1], vy)
                plsc.store_scatter(buf, [e + 2], vz)
                return carry

            lax.fori_loop(0, cnt // 16, sbody, 0)
            pltpu.sync_copy(buf.at[pl.ds(0, _C3)], out_hbm.at[b, pl.ds(k * _C3, _C3)])

    return sc_dropout


_NDMA = 8
_TOT = _B * _N * 3
_CHUNK = _TOT // _NDMA


@functools.cache
def _build_tc_copy():
    def body(pc_ref, out_ref, *sems):
        copies = [
            pltpu.make_async_copy(
                pc_ref.at[pl.ds(i * _CHUNK, _CHUNK)],
                out_ref.at[pl.ds(i * _CHUNK, _CHUNK)],
                sems[i],
            )
            for i in range(_NDMA)
        ]
        for c in copies:
            c.start()
        for c in copies:
            c.wait()

    return pl.pallas_call(
        body,
        out_shape=jax.ShapeDtypeStruct((_TOT,), jnp.float32),
        in_specs=[pl.BlockSpec(memory_space=pl.ANY)],
        out_specs=pl.BlockSpec(memory_space=pl.ANY),
        scratch_shapes=[pltpu.SemaphoreType.DMA] * _NDMA,
    )


def kernel(pc):
    if _HAS_DROPS:
        pc2 = pc.reshape(_B, _N * 3)
        out = _build_sc_dropout()(pc2, jnp.asarray(_IDX_NP), jnp.asarray(_TAB_NP))
        return out.reshape(_B, _N, 3)
    out = _build_tc_copy()(pc.reshape(_TOT))
    return out.reshape(_B, _N, 3)


# speedup vs baseline: 1.0753x; 1.0753x over previous
import functools

import jax
import jax.numpy as jnp
import numpy as np
from jax import lax
from jax.experimental import pallas as pl
from jax.experimental.pallas import tpu as pltpu
from jax.experimental.pallas import tpu_sc as plsc

_P = 0.7
_MAX_DROPOUT_RATIO = 0.6
_B, _N = 32, 65536

_key = jax.random.key(42)
_key, _kgate = jax.random.split(_key)
_GATE_V = float(jax.random.uniform(_kgate, (1,))[0])
_DROPS = []
for _b_i in range(_B):
    _key, _k1, _k2 = jax.random.split(_key, 3)
    _ratio = float(jax.random.uniform(_k1, (1,))[0]) * _MAX_DROPOUT_RATIO
    _num = int(_ratio * _N)
    if _GATE_V < _P and _num > 0:
        _perm = jax.random.permutation(_k2, _N)[:_num]
        _DROPS.append(np.sort(np.asarray(_perm, dtype=np.int32)))
    else:
        _DROPS.append(np.zeros((0,), dtype=np.int32))

_C = 8192
_NCH = _N // _C
_PAD_ROW = _C

_cnt = np.zeros((_B, _NCH), dtype=np.int32)
_off = np.zeros((_B, _NCH), dtype=np.int32)
_seqs = []
for _b_i in range(_B):
    parts = []
    pos = 0
    d = _DROPS[_b_i]
    for _k_i in range(_NCH):
        loc = d[(d >= _k_i * _C) & (d < (_k_i + 1) * _C)] - _k_i * _C
        n = loc.shape[0]
        npad = (-n) % 16
        if npad:
            loc = np.concatenate([loc, np.full((npad,), _PAD_ROW, np.int32)])
        _cnt[_b_i, _k_i] = loc.shape[0]
        _off[_b_i, _k_i] = pos
        pos += loc.shape[0]
        parts.append(loc)
    _seqs.append(np.concatenate(parts) if parts else np.zeros((0,), np.int32))

_LMAX = max(16, max(s.shape[0] for s in _seqs))
_LMAX += (-_LMAX) % 8
_IDX_NP = np.full((_B, _LMAX), _PAD_ROW, dtype=np.int32)
for _b_i in range(_B):
    _IDX_NP[_b_i, : _seqs[_b_i].shape[0]] = _seqs[_b_i]
_TAB_NP = np.zeros((_B, 16), dtype=np.int32)
_TAB_NP[:, :_NCH] = _cnt
_TAB_NP[:, 8 : 8 + _NCH] = _off

_NC, _NS = 2, 16

_HAS_DROPS = any(s.shape[0] for s in _seqs)

_C3 = _C * 3


@functools.cache
def _build_sc_dropout():
    mesh = plsc.VectorSubcoreMesh(
        core_axis_name="c", subcore_axis_name="s", num_cores=_NC, num_subcores=_NS
    )

    @functools.partial(
        pl.kernel,
        out_type=jax.ShapeDtypeStruct((_B, _N * 3), jnp.float32),
        mesh=mesh,
        compiler_params=pltpu.CompilerParams(
            use_tc_tiling_on_sc=False, needs_layout_passes=False
        ),
        scratch_types=[
            pltpu.VMEM((_C3 + 8,), jnp.float32),
            pltpu.VMEM((_LMAX,), jnp.int32),
            pltpu.VMEM((16,), jnp.int32),
        ],
    )
    def sc_dropout(pc_hbm, idx_hbm, tab_hbm, out_hbm, buf, idxv, tabv):
        b = lax.axis_index("s") * _NC + lax.axis_index("c")
        pltpu.sync_copy(idx_hbm.at[b], idxv)
        pltpu.sync_copy(tab_hbm.at[b], tabv)
        tv = tabv[pl.ds(0, 16)]
        vx = vy = vz = None
        for k in range(_NCH):
            pltpu.sync_copy(pc_hbm.at[b, pl.ds(k * _C3, _C3)], buf.at[pl.ds(0, _C3)])
            if k == 0:
                r0 = buf[pl.ds(0, 16)]
                vx = jnp.full((16,), r0[0], jnp.float32)
                vy = jnp.full((16,), r0[1], jnp.float32)
                vz = jnp.full((16,), r0[2], jnp.float32)
            cnt = tv[k]
            off = tv[8 + k]

            def sbody(i, carry, off=off, vx=vx, vy=vy, vz=vz):
                rows = idxv[pl.ds(off + i * 16, 16)]
                e = rows * 3
                plsc.store_scatter(buf, [e], vx)
                plsc.store_scatter(buf, [e + 1], vy)
                plsc.store_scatter(buf, [e + 2], vz)
                return carry

            lax.fori_loop(0, cnt // 16, sbody, 0)
            pltpu.sync_copy(buf.at[pl.ds(0, _C3)], out_hbm.at[b, pl.ds(k * _C3, _C3)])

    return sc_dropout


_TOT = _B * _N * 3
_ROWS, _COLS = 6144, 1024
_BROWS = 512


@functools.cache
def _build_tc_copy():
    def body(pc_ref, out_ref):
        out_ref[...] = pc_ref[...]

    return pl.pallas_call(
        body,
        grid=(_ROWS // _BROWS,),
        in_specs=[pl.BlockSpec((_BROWS, _COLS), lambda i: (i, 0))],
        out_specs=pl.BlockSpec((_BROWS, _COLS), lambda i: (i, 0)),
        out_shape=jax.ShapeDtypeStruct((_ROWS, _COLS), jnp.float32),
    )


def kernel(pc):
    if _HAS_DROPS:
        pc2 = pc.reshape(_B, _N * 3)
        out = _build_sc_dropout()(pc2, jnp.asarray(_IDX_NP), jnp.asarray(_TAB_NP))
        return out.reshape(_B, _N, 3)
    out = _build_tc_copy()(pc.reshape(_ROWS, _COLS))
    return out.reshape(_B, _N, 3)


# speedup vs baseline: 17.1573x; 15.9560x over previous
import functools

import jax
import jax.numpy as jnp
import numpy as np
from jax import lax
from jax.experimental import pallas as pl
from jax.experimental.pallas import tpu as pltpu
from jax.experimental.pallas import tpu_sc as plsc

_P = 0.7
_MAX_DROPOUT_RATIO = 0.6
_B, _N = 32, 65536

_key = jax.random.key(42)
_key, _kgate = jax.random.split(_key)
_GATE_V = float(jax.random.uniform(_kgate, (1,))[0])
_DROPS = []
for _b_i in range(_B):
    _key, _k1, _k2 = jax.random.split(_key, 3)
    _ratio = float(jax.random.uniform(_k1, (1,))[0]) * _MAX_DROPOUT_RATIO
    _num = int(_ratio * _N)
    if _GATE_V < _P and _num > 0:
        _perm = jax.random.permutation(_k2, _N)[:_num]
        _DROPS.append(np.sort(np.asarray(_perm, dtype=np.int32)))
    else:
        _DROPS.append(np.zeros((0,), dtype=np.int32))

_C = 8192
_NCH = _N // _C
_PAD_ROW = _C

_cnt = np.zeros((_B, _NCH), dtype=np.int32)
_off = np.zeros((_B, _NCH), dtype=np.int32)
_seqs = []
for _b_i in range(_B):
    parts = []
    pos = 0
    d = _DROPS[_b_i]
    for _k_i in range(_NCH):
        loc = d[(d >= _k_i * _C) & (d < (_k_i + 1) * _C)] - _k_i * _C
        n = loc.shape[0]
        npad = (-n) % 16
        if npad:
            loc = np.concatenate([loc, np.full((npad,), _PAD_ROW, np.int32)])
        _cnt[_b_i, _k_i] = loc.shape[0]
        _off[_b_i, _k_i] = pos
        pos += loc.shape[0]
        parts.append(loc)
    _seqs.append(np.concatenate(parts) if parts else np.zeros((0,), np.int32))

_LMAX = max(16, max(s.shape[0] for s in _seqs))
_LMAX += (-_LMAX) % 8
_IDX_NP = np.full((_B, _LMAX), _PAD_ROW, dtype=np.int32)
for _b_i in range(_B):
    _IDX_NP[_b_i, : _seqs[_b_i].shape[0]] = _seqs[_b_i]
_TAB_NP = np.zeros((_B, 16), dtype=np.int32)
_TAB_NP[:, :_NCH] = _cnt
_TAB_NP[:, 8 : 8 + _NCH] = _off

_NC, _NS = 2, 16

_HAS_DROPS = any(s.shape[0] for s in _seqs)

_C3 = _C * 3


@functools.cache
def _build_sc_dropout():
    mesh = plsc.VectorSubcoreMesh(
        core_axis_name="c", subcore_axis_name="s", num_cores=_NC, num_subcores=_NS
    )

    @functools.partial(
        pl.kernel,
        out_type=jax.ShapeDtypeStruct((_B, _N * 3), jnp.float32),
        mesh=mesh,
        compiler_params=pltpu.CompilerParams(
            use_tc_tiling_on_sc=False, needs_layout_passes=False
        ),
        scratch_types=[
            pltpu.VMEM((_C3 + 8,), jnp.float32),
            pltpu.VMEM((_LMAX,), jnp.int32),
            pltpu.VMEM((16,), jnp.int32),
        ],
    )
    def sc_dropout(pc_hbm, idx_hbm, tab_hbm, out_hbm, buf, idxv, tabv):
        b = lax.axis_index("s") * _NC + lax.axis_index("c")
        pltpu.sync_copy(idx_hbm.at[b], idxv)
        pltpu.sync_copy(tab_hbm.at[b], tabv)
        tv = tabv[pl.ds(0, 16)]
        vx = vy = vz = None
        for k in range(_NCH):
            pltpu.sync_copy(pc_hbm.at[b, pl.ds(k * _C3, _C3)], buf.at[pl.ds(0, _C3)])
            if k == 0:
                r0 = buf[pl.ds(0, 16)]
                vx = jnp.full((16,), r0[0], jnp.float32)
                vy = jnp.full((16,), r0[1], jnp.float32)
                vz = jnp.full((16,), r0[2], jnp.float32)
            cnt = tv[k]
            off = tv[8 + k]

            def sbody(i, carry, off=off, vx=vx, vy=vy, vz=vz):
                rows = idxv[pl.ds(off + i * 16, 16)]
                e = rows * 3
                plsc.store_scatter(buf, [e], vx)
                plsc.store_scatter(buf, [e + 1], vy)
                plsc.store_scatter(buf, [e + 2], vz)
                return carry

            lax.fori_loop(0, cnt // 16, sbody, 0)
            pltpu.sync_copy(buf.at[pl.ds(0, _C3)], out_hbm.at[b, pl.ds(k * _C3, _C3)])

    return sc_dropout


_E = _N * 3
_BCOLS = 16384


@functools.cache
def _build_tc_copy():
    def body(pc_ref, out_ref):
        out_ref[...] = pc_ref[...]

    return pl.pallas_call(
        body,
        grid=(_E // _BCOLS,),
        in_specs=[pl.BlockSpec((_B, _BCOLS), lambda i: (0, i))],
        out_specs=pl.BlockSpec((_B, _BCOLS), lambda i: (0, i)),
        out_shape=jax.ShapeDtypeStruct((_B, _E), jnp.float32),
    )


def kernel(pc):
    if _HAS_DROPS:
        pc2 = pc.reshape(_B, _E)
        out = _build_sc_dropout()(pc2, jnp.asarray(_IDX_NP), jnp.asarray(_TAB_NP))
        return out.reshape(_B, _N, 3)
    out = _build_tc_copy()(pc.reshape(_B, _E))
    return out.reshape(_B, _N, 3)


# speedup vs baseline: 463.9127x; 27.0388x over previous
import functools

import jax
import jax.numpy as jnp
import numpy as np
from jax import lax
from jax.experimental import pallas as pl
from jax.experimental.pallas import tpu as pltpu
from jax.experimental.pallas import tpu_sc as plsc

_P = 0.7
_MAX_DROPOUT_RATIO = 0.6
_B, _N = 32, 65536

_key = jax.random.key(42)
_key, _kgate = jax.random.split(_key)
_GATE_V = float(jax.random.uniform(_kgate, (1,))[0])
_DROPS = []
for _b_i in range(_B):
    _key, _k1, _k2 = jax.random.split(_key, 3)
    _ratio = float(jax.random.uniform(_k1, (1,))[0]) * _MAX_DROPOUT_RATIO
    _num = int(_ratio * _N)
    if _GATE_V < _P and _num > 0:
        _perm = jax.random.permutation(_k2, _N)[:_num]
        _DROPS.append(np.sort(np.asarray(_perm, dtype=np.int32)))
    else:
        _DROPS.append(np.zeros((0,), dtype=np.int32))

_C = 8192
_NCH = _N // _C
_PAD_ROW = _C

_cnt = np.zeros((_B, _NCH), dtype=np.int32)
_off = np.zeros((_B, _NCH), dtype=np.int32)
_seqs = []
for _b_i in range(_B):
    parts = []
    pos = 0
    d = _DROPS[_b_i]
    for _k_i in range(_NCH):
        loc = d[(d >= _k_i * _C) & (d < (_k_i + 1) * _C)] - _k_i * _C
        n = loc.shape[0]
        npad = (-n) % 16
        if npad:
            loc = np.concatenate([loc, np.full((npad,), _PAD_ROW, np.int32)])
        _cnt[_b_i, _k_i] = loc.shape[0]
        _off[_b_i, _k_i] = pos
        pos += loc.shape[0]
        parts.append(loc)
    _seqs.append(np.concatenate(parts) if parts else np.zeros((0,), np.int32))

_LMAX = max(16, max(s.shape[0] for s in _seqs))
_LMAX += (-_LMAX) % 8
_IDX_NP = np.full((_B, _LMAX), _PAD_ROW, dtype=np.int32)
for _b_i in range(_B):
    _IDX_NP[_b_i, : _seqs[_b_i].shape[0]] = _seqs[_b_i]
_TAB_NP = np.zeros((_B, 16), dtype=np.int32)
_TAB_NP[:, :_NCH] = _cnt
_TAB_NP[:, 8 : 8 + _NCH] = _off

_NC, _NS = 2, 16

_HAS_DROPS = any(s.shape[0] for s in _seqs)

_C3 = _C * 3


@functools.cache
def _build_sc_dropout():
    mesh = plsc.VectorSubcoreMesh(
        core_axis_name="c", subcore_axis_name="s", num_cores=_NC, num_subcores=_NS
    )

    @functools.partial(
        pl.kernel,
        out_type=jax.ShapeDtypeStruct((_B, _N * 3), jnp.float32),
        mesh=mesh,
        compiler_params=pltpu.CompilerParams(
            use_tc_tiling_on_sc=False, needs_layout_passes=False
        ),
        scratch_types=[
            pltpu.VMEM((_C3 + 8,), jnp.float32),
            pltpu.VMEM((_LMAX,), jnp.int32),
            pltpu.VMEM((16,), jnp.int32),
        ],
    )
    def sc_dropout(pc_hbm, idx_hbm, tab_hbm, out_hbm, buf, idxv, tabv):
        b = lax.axis_index("s") * _NC + lax.axis_index("c")
        pltpu.sync_copy(idx_hbm.at[b], idxv)
        pltpu.sync_copy(tab_hbm.at[b], tabv)
        tv = tabv[pl.ds(0, 16)]
        vx = vy = vz = None
        for k in range(_NCH):
            pltpu.sync_copy(pc_hbm.at[b, pl.ds(k * _C3, _C3)], buf.at[pl.ds(0, _C3)])
            if k == 0:
                r0 = buf[pl.ds(0, 16)]
                vx = jnp.full((16,), r0[0], jnp.float32)
                vy = jnp.full((16,), r0[1], jnp.float32)
                vz = jnp.full((16,), r0[2], jnp.float32)
            cnt = tv[k]
            off = tv[8 + k]

            def sbody(i, carry, off=off, vx=vx, vy=vy, vz=vz):
                rows = idxv[pl.ds(off + i * 16, 16)]
                e = rows * 3
                plsc.store_scatter(buf, [e], vx)
                plsc.store_scatter(buf, [e + 1], vy)
                plsc.store_scatter(buf, [e + 2], vz)
                return carry

            lax.fori_loop(0, cnt // 16, sbody, 0)
            pltpu.sync_copy(buf.at[pl.ds(0, _C3)], out_hbm.at[b, pl.ds(k * _C3, _C3)])

    return sc_dropout


_BCOLS = 16384


@functools.cache
def _build_tc_copy():
    def body(pc_ref, out_ref):
        out_ref[...] = pc_ref[...]

    return pl.pallas_call(
        body,
        grid=(3, _N // _BCOLS),
        in_specs=[pl.BlockSpec((1, _B, _BCOLS), lambda p, j: (p, 0, j))],
        out_specs=pl.BlockSpec((1, _B, _BCOLS), lambda p, j: (p, 0, j)),
        out_shape=jax.ShapeDtypeStruct((3, _B, _N), jnp.float32),
    )


def kernel(pc):
    if _HAS_DROPS:
        pc2 = pc.reshape(_B, _N * 3)
        out = _build_sc_dropout()(pc2, jnp.asarray(_IDX_NP), jnp.asarray(_TAB_NP))
        return out.reshape(_B, _N, 3)
    pct = jnp.transpose(pc, (2, 0, 1))
    out = _build_tc_copy()(pct)
    return jnp.transpose(out, (1, 2, 0))
